# baseline (speedup 1.0000x reference)
import functools

import jax
import jax.numpy as jnp
from jax import lax
from jax.experimental import pallas as pl
from jax.experimental.pallas import tpu as pltpu
from jax.experimental.pallas import tpu_sc as plsc

N = 50000
E = 200000
EA = 25000
FH = 64
FA = 16
HID = 64

_EB = 512
_EAP = 25088
_NP = 50176

_NC = 2
_NS = 16
_NW = _NC * _NS
_CH = 112


def _sc_mesh():
    return plsc.VectorSubcoreMesh(core_axis_name="c", subcore_axis_name="s")


def _sc_gather_body(bpw, table, idx_hbm, out_hbm, idx_v, rows_v, sem):
    wid = lax.axis_index("s") * _NC + lax.axis_index("c")
    base = wid * bpw
    nch = bpw // _CH
    for k in range(nch):
        pltpu.sync_copy(idx_hbm.at[pl.ds(base + k * _CH, _CH)], idx_v.at[k])
    hs = [pltpu.async_copy(table.at[idx_v.at[k]],
                           rows_v.at[pl.ds(k * _CH, _CH)], sem)
          for k in range(nch)]
    for h in hs:
        h.wait()
    pltpu.sync_copy(rows_v, out_hbm.at[pl.ds(base, bpw)])


def _sc_gather_rows(table, idx):
    b, d = idx.shape[0], table.shape[1]
    bpw = b // _NW
    assert bpw % _CH == 0
    return pl.kernel(
        functools.partial(_sc_gather_body, bpw),
        out_type=jax.ShapeDtypeStruct((b, d), table.dtype),
        mesh=_sc_mesh(),
        scratch_types=[
            pltpu.VMEM((bpw // _CH, _CH), jnp.int32),
            pltpu.VMEM((bpw, d), table.dtype),
            pltpu.SemaphoreType.DMA,
        ],
        compiler_params=pltpu.CompilerParams(use_tc_tiling_on_sc=False),
    )(table, idx)


_FQ = FH // 4
_TRASH = N


def _zero_rows16(ref, nrows):
    z16 = jnp.zeros((16,), jnp.float32)

    def _body(i, _):
        ref[i, pl.ds(0, 16)] = z16
        return _

    lax.fori_loop(0, nrows, _body, None)


def _sc_scatter_body(mq0, mq1, mq2, mq3, dst_hbm, o0, o1, o2, o3,
                     idx_v, rows_v, acc_sh, sem):
    c = lax.axis_index("c")
    s = lax.axis_index("s")
    ept = _EAP // _NS
    nch = ept // _CH
    zrows = _NP // _NS

    if True:
        base = s * ept
        for k in range(nch):
            pltpu.sync_copy(dst_hbm.at[pl.ds(base + k * _CH, _CH)], idx_v.at[k])
        for i in range(nch):
            for j in range(_CH // 16):
                off = i * _CH + j * 16
                d_v = idx_v[i, pl.ds(j * 16, 16)]
                eid = lax.iota(jnp.int32, 16) + (base + off)
                idx_v[i, pl.ds(j * 16, 16)] = jnp.where(eid < EA, d_v, _TRASH)

        ins = ((mq0, mq2), (mq1, mq3))
        outs = ((o0, o2), (o1, o3))
        for p in range(2):
            _zero_rows16(rows_v, ept)
            plsc.subcore_barrier()
            for r in range(zrows // ept):
                pltpu.sync_copy(rows_v, acc_sh.at[pl.ds(s * zrows + r * ept, ept)])
            plsc.subcore_barrier()

            @pl.when(c == 0)
            def _():
                pltpu.sync_copy(ins[p][0].at[pl.ds(base, ept)], rows_v)

            @pl.when(c == 1)
            def _():
                pltpu.sync_copy(ins[p][1].at[pl.ds(base, ept)], rows_v)

            hs = [pltpu.async_copy(rows_v.at[pl.ds(k * _CH, _CH)],
                                   acc_sh.at[idx_v.at[k]], sem, add=True)
                  for k in range(nch)]
            for h in hs:
                h.wait()
            plsc.subcore_barrier()

            @pl.when(c == 0)
            def _():
                pltpu.sync_copy(acc_sh.at[pl.ds(s * zrows, zrows)],
                                outs[p][0].at[pl.ds(s * zrows, zrows)])

            @pl.when(c == 1)
            def _():
                pltpu.sync_copy(acc_sh.at[pl.ds(s * zrows, zrows)],
                                outs[p][1].at[pl.ds(s * zrows, zrows)])

            plsc.subcore_barrier()


def _sc_scatter_add(mq0, mq1, mq2, mq3, dst):
    ept = _EAP // _NS
    st = jax.ShapeDtypeStruct((_NP, _FQ), jnp.float32)
    return pl.kernel(
        _sc_scatter_body,
        out_type=[st, st, st, st],
        mesh=_sc_mesh(),
        scratch_types=[
            pltpu.VMEM((ept // _CH, _CH), jnp.int32),
            pltpu.VMEM((ept, _FQ), jnp.float32),
            pltpu.VMEM_SHARED((_NP, _FQ), jnp.float32),
            pltpu.SemaphoreType.DMA,
        ],
        compiler_params=pltpu.CompilerParams(use_tc_tiling_on_sc=False),
    )(mq0, mq1, mq2, mq3, dst)


def _nnconv_msg_body(z_ref, xs_ref, w1_ref, b1_ref, w2s_ref, b2r_ref,
                     lo_ref, hi_ref, q2_ref, q3_ref):
    z = z_ref[...]
    v = jnp.dot(z, w1_ref[...], preferred_element_type=jnp.float32) + b1_ref[...]
    h = jnp.where(v >= 0, v, 0.01 * v)
    xs = xs_ref[...]
    t2 = jnp.dot(xs, w2s_ref[...], preferred_element_type=jnp.float32)
    acc = jnp.dot(xs, b2r_ref[...], preferred_element_type=jnp.float32)
    for j in range(HID):
        acc += h[:, j][:, None] * t2[:, j * FH:(j + 1) * FH]
    lo_ref[...] = acc[:, 0 * _FQ:1 * _FQ]
    hi_ref[...] = acc[:, 1 * _FQ:2 * _FQ]
    q2_ref[...] = acc[:, 2 * _FQ:3 * _FQ]
    q3_ref[...] = acc[:, 3 * _FQ:4 * _FQ]


def _nnconv_msg(z_pad, xs_pad, W1, b1, W2s, B2r):
    grid = (_EAP // _EB,)
    return pl.pallas_call(
        _nnconv_msg_body,
        grid=grid,
        in_specs=[
            pl.BlockSpec((_EB, FA), lambda i: (i, 0)),
            pl.BlockSpec((_EB, FH), lambda i: (i, 0)),
            pl.BlockSpec((FA, HID), lambda i: (0, 0)),
            pl.BlockSpec((1, HID), lambda i: (0, 0)),
            pl.BlockSpec((FH, HID * FH), lambda i: (0, 0)),
            pl.BlockSpec((FH, FH), lambda i: (0, 0)),
        ],
        out_specs=[pl.BlockSpec((_EB, _FQ), lambda i: (i, 0))] * 4,
        out_shape=[jax.ShapeDtypeStruct((_EAP, _FQ), jnp.float32)] * 4,
    )(z_pad, xs_pad, W1, b1, W2s, B2r)


def _gat_dense(h, src, dst, W, a_s, a_d, b):
    n = h.shape[0]
    loop = jnp.arange(n, dtype=src.dtype)
    s = jnp.concatenate([src, loop])
    d = jnp.concatenate([dst, loop])
    hp = h @ W
    al_s = jnp.sum(hp * a_s, axis=-1)
    al_d = jnp.sum(hp * a_d, axis=-1)
    e = jax.nn.leaky_relu(al_s[s] + al_d[d], negative_slope=0.2)
    m = jax.ops.segment_max(e, d, num_segments=n)
    m = jnp.where(jnp.isfinite(m), m, 0.0)
    w = jnp.exp(e - m[d])
    den = jax.ops.segment_sum(w, d, num_segments=n)
    coef = w / (den[d] + 1e-16)
    out = jax.ops.segment_sum(coef[:, None] * hp[s], d, num_segments=n)
    return out + b


def kernel(x, z_alpha, alpha_indices, edge_index,
           W1, b1, W2, b2, W_root, b_nn,
           Wg1, as1, ad1, bg1, Wg2, as2, ad2, bg2):
    src_a = alpha_indices[0]
    dst_a = alpha_indices[1]
    W2s = W2.reshape(HID, FH, FH).transpose(1, 0, 2).reshape(FH, HID * FH)
    B2r = b2.reshape(FH, FH)

    src_a_pad = jnp.pad(src_a, (0, _EAP - EA))
    dst_a_pad = jnp.pad(dst_a, (0, _EAP - EA))
    xs_pad = _sc_gather_rows(x, src_a_pad)
    z_pad = jnp.pad(z_alpha, ((0, _EAP - EA), (0, 0)))
    mq = _nnconv_msg(z_pad, xs_pad, W1, b1.reshape(1, HID), W2s, B2r)

    sq = _sc_scatter_add(mq[0], mq[1], mq[2], mq[3], dst_a_pad)
    seg = jnp.concatenate([q[:N] for q in sq], axis=1)
    out = seg + x @ W_root + b_nn
    out = _gat_dense(out, edge_index[0], edge_index[1], Wg1, as1, ad1, bg1)
    out = _gat_dense(out, edge_index[0], edge_index[1], Wg2, as2, ad2, bg2)
    return out


# speedup vs baseline: 2.7526x; 2.7526x over previous
import functools

import jax
import jax.numpy as jnp
from jax import lax
from jax.experimental import pallas as pl
from jax.experimental.pallas import tpu as pltpu
from jax.experimental.pallas import tpu_sc as plsc

N = 50000
E = 200000
EA = 25000
FH = 64
FA = 16
HID = 64

_EB = 512
_EAP = 25088
_NP = 50176

_NC = 2
_NS = 16
_NW = _NC * _NS
_CH = 112


def _sc_mesh():
    return plsc.VectorSubcoreMesh(core_axis_name="c", subcore_axis_name="s")


def _sc_gather_body(bpw, table, idx_hbm, out_hbm, idx_v, rows_v, sem):
    wid = lax.axis_index("s") * _NC + lax.axis_index("c")
    base = wid * bpw
    nch = bpw // _CH
    for k in range(nch):
        pltpu.sync_copy(idx_hbm.at[pl.ds(base + k * _CH, _CH)], idx_v.at[k])
    hs = [pltpu.async_copy(table.at[idx_v.at[k]],
                           rows_v.at[pl.ds(k * _CH, _CH)], sem)
          for k in range(nch)]
    for h in hs:
        h.wait()
    pltpu.sync_copy(rows_v, out_hbm.at[pl.ds(base, bpw)])


def _sc_gather_rows(table, idx):
    b, d = idx.shape[0], table.shape[1]
    bpw = b // _NW
    assert bpw % _CH == 0
    return pl.kernel(
        functools.partial(_sc_gather_body, bpw),
        out_type=jax.ShapeDtypeStruct((b, d), table.dtype),
        mesh=_sc_mesh(),
        scratch_types=[
            pltpu.VMEM((bpw // _CH, _CH), jnp.int32),
            pltpu.VMEM((bpw, d), table.dtype),
            pltpu.SemaphoreType.DMA,
        ],
        compiler_params=pltpu.CompilerParams(use_tc_tiling_on_sc=False,
                                             needs_layout_passes=False),
    )(table, idx)


_FQ = FH // 4
_TRASH = N


def _zero_rows16(ref, nrows):
    z16 = jnp.zeros((16,), jnp.float32)

    def _body(i, _):
        ref[i, pl.ds(0, 16)] = z16
        return _

    lax.fori_loop(0, nrows, _body, None)


def _sc_scatter_body(mq0, mq1, mq2, mq3, dst_hbm, o0, o1, o2, o3,
                     idx_v, rows_v, acc_sh, sem):
    c = lax.axis_index("c")
    s = lax.axis_index("s")
    ept = _EAP // _NS
    nch = ept // _CH
    zrows = _NP // _NS

    if True:
        base = s * ept
        for k in range(nch):
            pltpu.sync_copy(dst_hbm.at[pl.ds(base + k * _CH, _CH)], idx_v.at[k])
        for i in range(nch):
            for j in range(_CH // 16):
                off = i * _CH + j * 16
                d_v = idx_v[i, pl.ds(j * 16, 16)]
                eid = lax.iota(jnp.int32, 16) + (base + off)
                idx_v[i, pl.ds(j * 16, 16)] = jnp.where(eid < EA, d_v, _TRASH)

        ins = ((mq0, mq2), (mq1, mq3))
        outs = ((o0, o2), (o1, o3))
        for p in range(2):
            _zero_rows16(rows_v, ept)
            plsc.subcore_barrier()
            for r in range(zrows // ept):
                pltpu.sync_copy(rows_v, acc_sh.at[pl.ds(s * zrows + r * ept, ept)])
            plsc.subcore_barrier()

            @pl.when(c == 0)
            def _():
                pltpu.sync_copy(ins[p][0].at[pl.ds(base, ept)], rows_v)

            @pl.when(c == 1)
            def _():
                pltpu.sync_copy(ins[p][1].at[pl.ds(base, ept)], rows_v)

            hs = [pltpu.async_copy(rows_v.at[pl.ds(k * _CH, _CH)],
                                   acc_sh.at[idx_v.at[k]], sem, add=True)
                  for k in range(nch)]
            for h in hs:
                h.wait()
            plsc.subcore_barrier()

            @pl.when(c == 0)
            def _():
                pltpu.sync_copy(acc_sh.at[pl.ds(s * zrows, zrows)],
                                outs[p][0].at[pl.ds(s * zrows, zrows)])

            @pl.when(c == 1)
            def _():
                pltpu.sync_copy(acc_sh.at[pl.ds(s * zrows, zrows)],
                                outs[p][1].at[pl.ds(s * zrows, zrows)])

            plsc.subcore_barrier()


def _sc_scatter_add(mq0, mq1, mq2, mq3, dst):
    ept = _EAP // _NS
    st = jax.ShapeDtypeStruct((_NP, _FQ), jnp.float32)
    return pl.kernel(
        _sc_scatter_body,
        out_type=[st, st, st, st],
        mesh=_sc_mesh(),
        scratch_types=[
            pltpu.VMEM((ept // _CH, _CH), jnp.int32),
            pltpu.VMEM((ept, _FQ), jnp.float32),
            pltpu.VMEM_SHARED((_NP, _FQ), jnp.float32),
            pltpu.SemaphoreType.DMA,
        ],
        compiler_params=pltpu.CompilerParams(use_tc_tiling_on_sc=False,
                                             needs_layout_passes=False),
    )(mq0, mq1, mq2, mq3, dst)


def _nnconv_msg_body(z_ref, xs_ref, w1_ref, b1_ref, w2s_ref, b2r_ref,
                     lo_ref, hi_ref, q2_ref, q3_ref):
    z = z_ref[...]
    v = jnp.dot(z, w1_ref[...], preferred_element_type=jnp.float32) + b1_ref[...]
    h = jnp.where(v >= 0, v, 0.01 * v)
    xs = xs_ref[...]
    t2 = jnp.dot(xs, w2s_ref[...], preferred_element_type=jnp.float32)
    acc = jnp.dot(xs, b2r_ref[...], preferred_element_type=jnp.float32)
    for j in range(HID):
        acc += h[:, j][:, None] * t2[:, j * FH:(j + 1) * FH]
    lo_ref[...] = acc[:, 0 * _FQ:1 * _FQ]
    hi_ref[...] = acc[:, 1 * _FQ:2 * _FQ]
    q2_ref[...] = acc[:, 2 * _FQ:3 * _FQ]
    q3_ref[...] = acc[:, 3 * _FQ:4 * _FQ]


def _nnconv_msg(z_pad, xs_pad, W1, b1, W2s, B2r):
    grid = (_EAP // _EB,)
    return pl.pallas_call(
        _nnconv_msg_body,
        grid=grid,
        in_specs=[
            pl.BlockSpec((_EB, FA), lambda i: (i, 0)),
            pl.BlockSpec((_EB, FH), lambda i: (i, 0)),
            pl.BlockSpec((FA, HID), lambda i: (0, 0)),
            pl.BlockSpec((1, HID), lambda i: (0, 0)),
            pl.BlockSpec((FH, HID * FH), lambda i: (0, 0)),
            pl.BlockSpec((FH, FH), lambda i: (0, 0)),
        ],
        out_specs=[pl.BlockSpec((_EB, _FQ), lambda i: (i, 0))] * 4,
        out_shape=[jax.ShapeDtypeStruct((_EAP, _FQ), jnp.float32)] * 4,
    )(z_pad, xs_pad, W1, b1, W2s, B2r)


_EP = 200704
_EPT1 = _EP // _NW
_EPT2 = _EP // _NS
_NEG = -3e38


def _lane_gather(v, idx):
    dn = lax.GatherDimensionNumbers(offset_dims=(), collapsed_slice_dims=(0,),
                                    start_index_map=(0,))
    return lax.gather(v, idx[:, None], dn, (1,),
                      mode=lax.GatherScatterMode.PROMISE_IN_BOUNDS)


def _rot(v, r):
    perm = (lax.iota(jnp.int32, 16) + r) & 15
    return _lane_gather(v, perm)


def _sc_gat_pass1_body(als_hbm, src_hbm, dst_hbm, qp_out, g_out,
                       als_v, q_v, srcb, dstb, gb, sem):
    wid = lax.axis_index("s") * _NC + lax.axis_index("c")
    base = wid * _EPT1
    pltpu.sync_copy(als_hbm, als_v)
    pltpu.sync_copy(src_hbm.at[pl.ds(base, _EPT1)], srcb)
    pltpu.sync_copy(dst_hbm.at[pl.ds(base, _EPT1)], dstb)

    neg = jnp.full((16,), _NEG, jnp.float32)

    def _init(i, _):
        q_v[pl.ds(i * 16, 16)] = neg
        return _

    lax.fori_loop(0, _NP // 16, _init, None)

    def _edge_group(i, _):
        off = i * 16
        s_v = srcb[pl.ds(off, 16)]
        d_v = dstb[pl.ds(off, 16)]
        g_v = plsc.load_gather(als_v, [s_v])
        gb[pl.ds(off, 16)] = g_v
        vv = g_v
        for r in range(1, 16):
            d_r = _rot(d_v, r)
            g_r = _rot(g_v, r)
            vv = jnp.where(d_v == d_r, jnp.maximum(vv, g_r), vv)
        cur = plsc.load_gather(q_v, [d_v])
        plsc.store_scatter(q_v, [d_v], jnp.maximum(vv, cur))
        return _

    lax.fori_loop(0, _EPT1 // 16, _edge_group, None)

    pltpu.sync_copy(gb, g_out.at[pl.ds(base, _EPT1)])
    pltpu.sync_copy(q_v, qp_out.at[pl.ds(wid * _NP, _NP)])


def _sc_gat_pass1(al_s, src, dst):
    return pl.kernel(
        _sc_gat_pass1_body,
        out_type=[jax.ShapeDtypeStruct((_NW * _NP,), jnp.float32),
                  jax.ShapeDtypeStruct((_EP,), jnp.float32)],
        mesh=_sc_mesh(),
        scratch_types=[
            pltpu.VMEM((_NP,), jnp.float32),
            pltpu.VMEM((_NP,), jnp.float32),
            pltpu.VMEM((_EPT1,), jnp.int32),
            pltpu.VMEM((_EPT1,), jnp.int32),
            pltpu.VMEM((_EPT1,), jnp.float32),
            pltpu.SemaphoreType.DMA,
        ],
        compiler_params=pltpu.CompilerParams(use_tc_tiling_on_sc=False,
                                             needs_layout_passes=False),
    )(al_s, src, dst)


_ACH = 11760
_ACR = 11776


def _sc_gat_pass2_body(g_hbm, dst_hbm, ald_hbm, m_hbm, hA, hB,
                       nA, nB, den_out,
                       ald_v, m_v, dstb2, dstbF, gb, wbuf, rows_v,
                       acc_sh, den_sh, sem):
    c = lax.axis_index("c")
    s = lax.axis_index("s")
    nch = 784 // _CH
    zrows = _NP // _NS

    pltpu.sync_copy(ald_hbm, ald_v)
    pltpu.sync_copy(m_hbm, m_v)

    z16 = jnp.zeros((16,), jnp.float32)

    def _zw(i, _):
        wbuf[pl.ds(i * 16, 16)] = z16
        return _

    lax.fori_loop(0, 784 // 16, _zw, None)
    for r in range(zrows // 784):
        pltpu.sync_copy(wbuf, den_sh.at[pl.ds(s * zrows + r * 784, 784)])

    for nh in range(5):
        nbase = nh * _ACH
        _zero_rows16(rows_v, 784)
        zr = _ACR // _NS
        pltpu.sync_copy(rows_v.at[pl.ds(0, zr)],
                        acc_sh.at[pl.ds(s * zr, zr)])
        plsc.subcore_barrier()

        def _chunk(ch, _):
            base = s * _EPT2 + ch * 784
            for k in range(nch):
                pltpu.sync_copy(dst_hbm.at[pl.ds(base + k * _CH, _CH)],
                                dstb2.at[k])
            pltpu.sync_copy(g_hbm.at[pl.ds(base, 784)], gb)

            def _wgrp(i, _):
                k = i // 7
                j = i - k * 7
                off = i * 16
                d_v = dstb2[k, pl.ds(j * 16, 16)]
                g_v = gb[pl.ds(off, 16)]
                ad = plsc.load_gather(ald_v, [d_v])
                mm = plsc.load_gather(m_v, [d_v])
                z = g_v + ad
                lr = jnp.maximum(z, 0.2 * z)
                wbuf[pl.ds(off, 16)] = jnp.exp(lr - mm)
                eid = lax.iota(jnp.int32, 16) + (base + off)
                dF = jnp.where(eid < E, d_v, _TRASH)
                dstbF[k, pl.ds(j * 16, 16)] = dF
                loc = d_v - nbase
                ok = (loc >= 0) & (loc < _ACH) & (eid < E)
                dstb2[k, pl.ds(j * 16, 16)] = jnp.where(ok, loc, _ACR - 1)
                return _

            lax.fori_loop(0, 784 // 16, _wgrp, None)

            @pl.when(c == 0)
            def _():
                pltpu.sync_copy(hA.at[pl.ds(base, 784)], rows_v)

            @pl.when(c == 1)
            def _():
                pltpu.sync_copy(hB.at[pl.ds(base, 784)], rows_v)

            def _scale(i, _):
                off = i * 16
                w_v = wbuf[pl.ds(off, 16)]
                for l in range(16):
                    b = _lane_gather(w_v, jnp.full((16,), l, jnp.int32))
                    rows_v[off + l, pl.ds(0, 16)] = (
                        rows_v[off + l, pl.ds(0, 16)] * b)
                return _

            lax.fori_loop(0, 784 // 16, _scale, None)

            hs = [pltpu.async_copy(rows_v.at[pl.ds(k * _CH, _CH)],
                                   acc_sh.at[dstb2.at[k]], sem, add=True)
                  for k in range(nch)]
            for h in hs:
                h.wait()

            if nh == 0:
                @pl.when(c == 0)
                def _():
                    hs = [pltpu.async_copy(wbuf.at[pl.ds(k * _CH, _CH)],
                                           den_sh.at[dstbF.at[k]], sem,
                                           add=True)
                          for k in range(nch)]
                    for h in hs:
                        h.wait()
            return _

        lax.fori_loop(0, _EPT2 // 784, _chunk, None)
        plsc.subcore_barrier()

        orows = _ACH // _NS

        @pl.when(c == 0)
        def _():
            pltpu.sync_copy(acc_sh.at[pl.ds(s * orows, orows)],
                            nA.at[pl.ds(nbase + s * orows, orows)])

        @pl.when(c == 1)
        def _():
            pltpu.sync_copy(acc_sh.at[pl.ds(s * orows, orows)],
                            nB.at[pl.ds(nbase + s * orows, orows)])

        plsc.subcore_barrier()

    @pl.when(c == 0)
    def _():
        pltpu.sync_copy(den_sh.at[pl.ds(s * zrows, zrows)],
                        den_out.at[pl.ds(s * zrows, zrows)])


def _sc_gat_pass2(g, dst, al_d, m, ghA, ghB):
    stq = jax.ShapeDtypeStruct((_NP, _FQ), jnp.float32)
    return pl.kernel(
        _sc_gat_pass2_body,
        out_type=[stq, stq, jax.ShapeDtypeStruct((_NP,), jnp.float32)],
        mesh=_sc_mesh(),
        scratch_types=[
            pltpu.VMEM((_NP,), jnp.float32),
            pltpu.VMEM((_NP,), jnp.float32),
            pltpu.VMEM((784 // _CH, _CH), jnp.int32),
            pltpu.VMEM((784 // _CH, _CH), jnp.int32),
            pltpu.VMEM((784,), jnp.float32),
            pltpu.VMEM((784,), jnp.float32),
            pltpu.VMEM((784, _FQ), jnp.float32),
            pltpu.VMEM_SHARED((_ACR, _FQ), jnp.float32),
            pltpu.VMEM_SHARED((_NP,), jnp.float32),
            pltpu.SemaphoreType.DMA,
        ],
        compiler_params=pltpu.CompilerParams(use_tc_tiling_on_sc=False,
                                             needs_layout_passes=False),
    )(g, dst, al_d, m, ghA, ghB)


_BN = 1024


def _tc_hp_al_body(s0, s1, s2, s3, xp, wg, wr, brow, asr, adr,
                   h0, h1, h2, h3, als, ald):
    S = jnp.concatenate([s0[...], s1[...], s2[...], s3[...]], axis=1)
    hp = (jnp.dot(S, wg[...], preferred_element_type=jnp.float32)
          + jnp.dot(xp[...], wr[...], preferred_element_type=jnp.float32)
          + brow[...])
    h0[...] = hp[:, 0 * _FQ:1 * _FQ]
    h1[...] = hp[:, 1 * _FQ:2 * _FQ]
    h2[...] = hp[:, 2 * _FQ:3 * _FQ]
    h3[...] = hp[:, 3 * _FQ:4 * _FQ]
    als[...] = jnp.sum(hp * asr[...], axis=1)
    ald[...] = jnp.sum(hp * adr[...], axis=1)


def _tc_hp_al(sq, xp, Wg, WrG, brow, a_s, a_d):
    stq = jax.ShapeDtypeStruct((_NP, _FQ), jnp.float32)
    st1 = jax.ShapeDtypeStruct((_NP,), jnp.float32)
    bq = pl.BlockSpec((_BN, _FQ), lambda i: (i, 0))
    b1 = pl.BlockSpec((_BN,), lambda i: (i,))
    full = lambda shape: pl.BlockSpec(shape, lambda i: tuple(0 for _ in shape))
    return pl.pallas_call(
        _tc_hp_al_body,
        grid=(_NP // _BN,),
        in_specs=[bq, bq, bq, bq,
                  pl.BlockSpec((_BN, FH), lambda i: (i, 0)),
                  full((FH, FH)), full((FH, FH)), full((1, FH)),
                  full((1, FH)), full((1, FH))],
        out_specs=[bq, bq, bq, bq, b1, b1],
        out_shape=[stq, stq, stq, stq, st1, st1],
    )(sq[0], sq[1], sq[2], sq[3], xp, Wg, WrG, brow, a_s, a_d)


def _tc_seg_max_body(qp, als, ald, m_ref):
    q = jnp.max(qp[...], axis=0)
    q2 = jnp.maximum(q, als[...])
    z = q2 + ald[...]
    m_ref[...] = jnp.maximum(z, 0.2 * z)


def _tc_seg_max(qp2, als, ald):
    b1 = pl.BlockSpec((_BN,), lambda i: (i,))
    return pl.pallas_call(
        _tc_seg_max_body,
        grid=(_NP // _BN,),
        in_specs=[pl.BlockSpec((_NW, _BN), lambda i: (0, i)), b1, b1],
        out_specs=b1,
        out_shape=jax.ShapeDtypeStruct((_NP,), jnp.float32),
    )(qp2, als, ald)


def _tc_combine_body(has_next, n0, n1, n2, n3, h0, h1, h2, h3,
                     als, ald, m, den, brow, *rest):
    hp = jnp.concatenate([h0[...], h1[...], h2[...], h3[...]], axis=1)
    numer = jnp.concatenate([n0[...], n1[...], n2[...], n3[...]], axis=1)
    z = als[...] + ald[...]
    lr = jnp.maximum(z, 0.2 * z)
    wself = jnp.exp(lr - m[...])
    dent = den[...] + wself + 1e-16
    outv = (numer + wself[:, None] * hp) / dent[:, None] + brow[...]
    if has_next:
        wg, asr, adr, o0, o1, o2, o3, als2, ald2 = rest
        hp2 = jnp.dot(outv, wg[...], preferred_element_type=jnp.float32)
        o0[...] = hp2[:, 0 * _FQ:1 * _FQ]
        o1[...] = hp2[:, 1 * _FQ:2 * _FQ]
        o2[...] = hp2[:, 2 * _FQ:3 * _FQ]
        o3[...] = hp2[:, 3 * _FQ:4 * _FQ]
        als2[...] = jnp.sum(hp2 * asr[...], axis=1)
        ald2[...] = jnp.sum(hp2 * adr[...], axis=1)
    else:
        (out_ref,) = rest
        out_ref[...] = outv


def _tc_combine_next(nq, hq, als, ald, m, den, brow, Wg2, a_s2, a_d2):
    stq = jax.ShapeDtypeStruct((_NP, _FQ), jnp.float32)
    st1 = jax.ShapeDtypeStruct((_NP,), jnp.float32)
    bq = pl.BlockSpec((_BN, _FQ), lambda i: (i, 0))
    b1 = pl.BlockSpec((_BN,), lambda i: (i,))
    full = lambda shape: pl.BlockSpec(shape, lambda i: tuple(0 for _ in shape))
    return pl.pallas_call(
        functools.partial(_tc_combine_body, True),
        grid=(_NP // _BN,),
        in_specs=[bq] * 8 + [b1, b1, b1, b1, full((1, FH)),
                             full((FH, FH)), full((1, FH)), full((1, FH))],
        out_specs=[bq, bq, bq, bq, b1, b1],
        out_shape=[stq, stq, stq, stq, st1, st1],
    )(nq[0], nq[1], nq[2], nq[3], hq[0], hq[1], hq[2], hq[3],
      als, ald, m, den, brow, Wg2, a_s2, a_d2)


def _tc_combine_final(nq, hq, als, ald, m, den, brow):
    bq = pl.BlockSpec((_BN, _FQ), lambda i: (i, 0))
    b1 = pl.BlockSpec((_BN,), lambda i: (i,))
    full = lambda shape: pl.BlockSpec(shape, lambda i: tuple(0 for _ in shape))
    return pl.pallas_call(
        functools.partial(_tc_combine_body, False),
        grid=(_NP // _BN,),
        in_specs=[bq] * 8 + [b1, b1, b1, b1, full((1, FH))],
        out_specs=pl.BlockSpec((_BN, FH), lambda i: (i, 0)),
        out_shape=jax.ShapeDtypeStruct((_NP, FH), jnp.float32),
    )(nq[0], nq[1], nq[2], nq[3], hq[0], hq[1], hq[2], hq[3],
      als, ald, m, den, brow)


def _gat_dense(h, src, dst, W, a_s, a_d, b):
    n = h.shape[0]
    loop = jnp.arange(n, dtype=src.dtype)
    s = jnp.concatenate([src, loop])
    d = jnp.concatenate([dst, loop])
    hp = h @ W
    al_s = jnp.sum(hp * a_s, axis=-1)
    al_d = jnp.sum(hp * a_d, axis=-1)
    e = jax.nn.leaky_relu(al_s[s] + al_d[d], negative_slope=0.2)
    m = jax.ops.segment_max(e, d, num_segments=n)
    m = jnp.where(jnp.isfinite(m), m, 0.0)
    w = jnp.exp(e - m[d])
    den = jax.ops.segment_sum(w, d, num_segments=n)
    coef = w / (den[d] + 1e-16)
    out = jax.ops.segment_sum(coef[:, None] * hp[s], d, num_segments=n)
    return out + b


def kernel(x, z_alpha, alpha_indices, edge_index,
           W1, b1, W2, b2, W_root, b_nn,
           Wg1, as1, ad1, bg1, Wg2, as2, ad2, bg2):
    src_a = alpha_indices[0]
    dst_a = alpha_indices[1]
    W2s = W2.reshape(HID, FH, FH).transpose(1, 0, 2).reshape(FH, HID * FH)
    B2r = b2.reshape(FH, FH)

    src_a_pad = jnp.pad(src_a, (0, _EAP - EA))
    dst_a_pad = jnp.pad(dst_a, (0, _EAP - EA))
    xs_pad = _sc_gather_rows(x, src_a_pad)
    z_pad = jnp.pad(z_alpha, ((0, _EAP - EA), (0, 0)))
    mq = _nnconv_msg(z_pad, xs_pad, W1, b1.reshape(1, HID), W2s, B2r)

    sq = _sc_scatter_add(mq[0], mq[1], mq[2], mq[3], dst_a_pad)

    xp = jnp.pad(x, ((0, _NP - N), (0, 0)))
    srcp = jnp.pad(edge_index[0], (0, _EP - E))
    dstp = jnp.pad(edge_index[1], (0, _EP - E))

    WrG1 = W_root @ Wg1
    brow1 = (b_nn @ Wg1).reshape(1, FH)
    r1 = _tc_hp_al(sq, xp, Wg1, WrG1, brow1,
                   as1.reshape(1, FH), ad1.reshape(1, FH))
    h1q, als1, ald1 = r1[:4], r1[4], r1[5]
    qflat1, g1 = _sc_gat_pass1(als1, srcp, dstp)
    m1 = _tc_seg_max(qflat1.reshape(_NW, _NP), als1, ald1)
    ghp1 = [_sc_gather_rows(h, srcp) for h in h1q]
    n1a, n1c, den1 = _sc_gat_pass2(g1, dstp, ald1, m1, ghp1[0], ghp1[2])
    n1b, n1d, _ = _sc_gat_pass2(g1, dstp, ald1, m1, ghp1[1], ghp1[3])
    nq1 = (n1a, n1b, n1c, n1d)

    r2 = _tc_combine_next(nq1, h1q, als1, ald1, m1, den1, bg1.reshape(1, FH),
                          Wg2, as2.reshape(1, FH), ad2.reshape(1, FH))
    h2q, als2, ald2 = r2[:4], r2[4], r2[5]
    qflat2, g2 = _sc_gat_pass1(als2, srcp, dstp)
    m2 = _tc_seg_max(qflat2.reshape(_NW, _NP), als2, ald2)
    ghp2 = [_sc_gather_rows(h, srcp) for h in h2q]
    n2a, n2c, den2 = _sc_gat_pass2(g2, dstp, ald2, m2, ghp2[0], ghp2[2])
    n2b, n2d, _ = _sc_gat_pass2(g2, dstp, ald2, m2, ghp2[1], ghp2[3])
    nq2 = (n2a, n2b, n2c, n2d)

    out = _tc_combine_final(nq2, h2q, als2, ald2, m2, den2, bg2.reshape(1, FH))
    return out[:N]
